# baseline (speedup 1.0000x reference)
import jax
import jax.numpy as jnp
from jax.experimental import pallas as pl


def _copy_body(inputs_ref, buffer_ref, out_ref):
    i = pl.program_id(0)

    @pl.when(i == 0)
    def _():
        out_ref[...] = inputs_ref[...]

    @pl.when(i > 0)
    def _():
        out_ref[...] = buffer_ref[...]


def kernel(inputs, buffer):
    b, d = inputs.shape
    n_steps = buffer.shape[0]
    seg = n_steps // b
    w = 128
    rows_in = b * d // w
    rows_all = n_steps * d // w
    n_blocks = rows_all // rows_in

    inputs2 = inputs.reshape(rows_in, w)
    buffer2 = buffer.reshape(rows_all, w)

    out2 = pl.pallas_call(
        _copy_body,
        grid=(n_blocks,),
        in_specs=[
            pl.BlockSpec((rows_in, w), lambda i: (0, 0)),
            pl.BlockSpec((rows_in, w), lambda i: (jnp.maximum(i - 1, 0), 0)),
        ],
        out_specs=pl.BlockSpec((rows_in, w), lambda i: (i, 0)),
        out_shape=jax.ShapeDtypeStruct((rows_all, w), inputs.dtype),
    )(inputs2, buffer2)
    return out2.reshape(b, seg, d)


# speedup vs baseline: 1.6216x; 1.6216x over previous
import jax
import jax.numpy as jnp
from jax.experimental import pallas as pl


def _copy_body(inputs_ref, buffer_ref, out_ref):
    i = pl.program_id(0)

    @pl.when(i == 0)
    def _():
        out_ref[...] = inputs_ref[...]

    @pl.when(i > 0)
    def _():
        out_ref[...] = buffer_ref[...]


def kernel(inputs, buffer):
    b, d = inputs.shape
    n_steps = buffer.shape[0]
    seg = n_steps // b
    sub = 8
    rows_in = b // sub
    rows_all = n_steps // sub
    n_blocks = rows_all // rows_in

    inputs3 = inputs.reshape(rows_in, sub, d)
    buffer3 = buffer.reshape(rows_all, sub, d)

    out3 = pl.pallas_call(
        _copy_body,
        grid=(n_blocks,),
        in_specs=[
            pl.BlockSpec((rows_in, sub, d), lambda i: (0, 0, 0)),
            pl.BlockSpec((rows_in, sub, d), lambda i: (jnp.maximum(i - 1, 0), 0, 0)),
        ],
        out_specs=pl.BlockSpec((rows_in, sub, d), lambda i: (i, 0, 0)),
        out_shape=jax.ShapeDtypeStruct((rows_all, sub, d), inputs.dtype),
    )(inputs3, buffer3)
    return out3.reshape(b, seg, d)
